# baseline (speedup 1.0000x reference)
import functools

import jax
import jax.numpy as jnp
from jax import lax
from jax.experimental import pallas as pl
from jax.experimental.pallas import tpu as pltpu
from jax.experimental.pallas import tpu_sc as plsc

_K = 2
_LBW = 0.01
_TM = 512
_HB = 1024
_TMG = 1024
_NW = 32
_GCH = 32
_CCH = 16


def _gate_body(x_ref, wg_ref, bg_ref, i1_ref, i2_ref, g1_ref, g2_ref, cnt_ref):
    e = wg_ref.shape[0]
    logits = jnp.dot(x_ref[...], wg_ref[...].T,
                     preferred_element_type=jnp.float32) + bg_ref[0]
    iota = lax.broadcasted_iota(jnp.int32, logits.shape, 1)
    m1 = jnp.max(logits, axis=1, keepdims=True)
    i1 = jnp.min(jnp.where(logits == m1, iota, e), axis=1, keepdims=True)
    masked = jnp.where(iota == i1, -jnp.inf, logits)
    m2 = jnp.max(masked, axis=1, keepdims=True)
    i2 = jnp.min(jnp.where(masked == m2, iota, e), axis=1, keepdims=True)
    e2 = jnp.exp(m2 - m1)
    den = 1.0 + e2
    i1_ref[...] = i1
    i2_ref[...] = i2
    g1_ref[...] = 1.0 / den
    g2_ref[...] = e2 / den
    cnt = jnp.sum((iota == i1).astype(jnp.int32) + (iota == i2).astype(jnp.int32),
                  axis=0)
    cnt_ref[...] = cnt.reshape(1, 1, e)


def _gating(xf, Wg, bg):
    t, d = xf.shape
    e = Wg.shape[0]
    nt = t // _TMG
    out_shapes = (
        jax.ShapeDtypeStruct((t, 1), jnp.int32),
        jax.ShapeDtypeStruct((t, 1), jnp.int32),
        jax.ShapeDtypeStruct((t, 1), jnp.float32),
        jax.ShapeDtypeStruct((t, 1), jnp.float32),
        jax.ShapeDtypeStruct((nt, 1, e), jnp.int32),
    )
    col = pl.BlockSpec((_TMG, 1), lambda i: (i, 0))
    return pl.pallas_call(
        _gate_body,
        grid=(nt,),
        in_specs=[
            pl.BlockSpec((_TMG, d), lambda i: (i, 0)),
            pl.BlockSpec((e, d), lambda i: (0, 0)),
            pl.BlockSpec((1, e), lambda i: (0, 0)),
        ],
        out_specs=(col, col, col, col,
                   pl.BlockSpec((1, 1, e), lambda i: (i, 0, 0))),
        out_shape=out_shapes,
    )(xf, Wg, bg.reshape(1, e))


def _make_gather(t, d, mp):
    rows_per_w = mp // _NW
    n_chunks = rows_per_w // _GCH
    mesh = plsc.VectorSubcoreMesh(core_axis_name="c", subcore_axis_name="s")

    @functools.partial(
        pl.kernel,
        mesh=mesh,
        out_type=jax.ShapeDtypeStruct((mp, d), jnp.float32),
        scratch_types=[
            pltpu.VMEM((_GCH,), jnp.int32),
            pltpu.VMEM((_GCH, d), jnp.float32),
            pltpu.SemaphoreType.DMA,
        ],
    )
    def gather(tab_hbm, idx_hbm, out_hbm, idx_v, rows_v, sem):
        wid = lax.axis_index("s") * 2 + lax.axis_index("c")
        base = wid * rows_per_w

        def chunk(c, carry):
            off = base + c * _GCH
            pltpu.sync_copy(idx_hbm.at[pl.ds(off, _GCH)], idx_v)
            pltpu.async_copy(tab_hbm.at[idx_v], rows_v, sem).wait()
            pltpu.sync_copy(rows_v, out_hbm.at[pl.ds(off, _GCH)])
            return carry

        lax.fori_loop(0, n_chunks, chunk, 0)

    return gather


def _gmm_body(e_ref, xg_ref, w1_ref, w2_ref, b1_ref, b2_ref, g_ref, y_ref,
              acc_ref):
    j = pl.program_id(1)

    @pl.when(j == 0)
    def _():
        acc_ref[...] = jnp.zeros_like(acc_ref)

    xb = xg_ref[...].astype(jnp.bfloat16)
    h = lax.dot_general(xb, w1_ref[0], (((1,), (1,)), ((), ())),
                        preferred_element_type=jnp.float32)
    h = jnp.maximum(h + b1_ref[0, 0], 0.0).astype(jnp.bfloat16)
    acc_ref[...] += lax.dot_general(h, w2_ref[0], (((1,), (1,)), ((), ())),
                                    preferred_element_type=jnp.float32)

    @pl.when(j == pl.num_programs(1) - 1)
    def _():
        y_ref[...] = (acc_ref[...] + b2_ref[0, 0]) * g_ref[...]


def _gmm(xg, W1, b1, W2, b2, row_gate, e_tile):
    mp, d = xg.shape
    e, h, _ = W1.shape
    do = W2.shape[1]
    n_tiles = mp // _TM
    nh = h // _HB
    grid_spec = pltpu.PrefetchScalarGridSpec(
        num_scalar_prefetch=1,
        grid=(n_tiles, nh),
        in_specs=[
            pl.BlockSpec((_TM, d), lambda i, j, er: (i, 0)),
            pl.BlockSpec((1, _HB, d), lambda i, j, er: (er[i], j, 0)),
            pl.BlockSpec((1, do, _HB), lambda i, j, er: (er[i], 0, j)),
            pl.BlockSpec((1, 1, 1, _HB), lambda i, j, er: (er[i], j, 0, 0)),
            pl.BlockSpec((1, 1, do), lambda i, j, er: (er[i], 0, 0)),
            pl.BlockSpec((_TM, 1), lambda i, j, er: (i, 0)),
        ],
        out_specs=pl.BlockSpec((_TM, do), lambda i, j, er: (i, 0)),
        scratch_shapes=[pltpu.VMEM((_TM, do), jnp.float32)],
    )
    return pl.pallas_call(
        _gmm_body,
        grid_spec=grid_spec,
        out_shape=jax.ShapeDtypeStruct((mp, do), jnp.float32),
    )(e_tile, xg, W1.astype(jnp.bfloat16), W2.astype(jnp.bfloat16),
      b1.reshape(e, h // _HB, 1, _HB), b2.reshape(e, 1, do), row_gate)


def _make_combine(t, do, mp):
    tok_per_w = t // _NW
    n_chunks = tok_per_w // _CCH
    nv = do // 16
    mesh = plsc.VectorSubcoreMesh(core_axis_name="c", subcore_axis_name="s")

    @functools.partial(
        pl.kernel,
        mesh=mesh,
        out_type=jax.ShapeDtypeStruct((t, do), jnp.float32),
        scratch_types=[
            pltpu.VMEM((_CCH,), jnp.int32),
            pltpu.VMEM((_CCH,), jnp.int32),
            pltpu.VMEM((_CCH, do), jnp.float32),
            pltpu.VMEM((_CCH, do), jnp.float32),
            pltpu.SemaphoreType.DMA,
            pltpu.SemaphoreType.DMA,
        ],
    )
    def combine(y_hbm, p0_hbm, p1_hbm, out_hbm, ia_v, ib_v, ra_v, rb_v,
                sem_a, sem_b):
        wid = lax.axis_index("s") * 2 + lax.axis_index("c")
        base = wid * tok_per_w

        def chunk(c, carry):
            toff = base + c * _CCH
            pltpu.sync_copy(p0_hbm.at[pl.ds(toff, _CCH)], ia_v)
            pltpu.sync_copy(p1_hbm.at[pl.ds(toff, _CCH)], ib_v)
            cp_a = pltpu.async_copy(y_hbm.at[ia_v], ra_v, sem_a)
            cp_b = pltpu.async_copy(y_hbm.at[ib_v], rb_v, sem_b)
            cp_a.wait()
            cp_b.wait()
            for r in range(_CCH):
                def add_col(v, carry2):
                    sl = pl.ds(v * 16, 16)
                    ra_v[r, sl] = ra_v[r, sl] + rb_v[r, sl]
                    return carry2
                lax.fori_loop(0, nv, add_col, 0)
            pltpu.sync_copy(ra_v, out_hbm.at[pl.ds(toff, _CCH)])
            return carry

        lax.fori_loop(0, n_chunks, chunk, 0)

    return combine


def kernel(x, Wg, bg, W1, b1, W2, b2):
    b, s, d = x.shape
    t = b * s
    e, h, _ = W1.shape
    do = W2.shape[1]
    mp = t * _K + e * _TM

    xf = x.reshape(t, d)
    i1, i2, g1, g2, cnt_tiles = _gating(xf, Wg, bg)
    counts = jnp.sum(cnt_tiles, axis=(0, 1))

    expert_flat = jnp.concatenate([i1, i2], axis=1).reshape(-1)
    gate_flat = jnp.concatenate([g1, g2], axis=1).reshape(-1)
    token_flat = jnp.repeat(jnp.arange(t, dtype=jnp.int32), _K)
    order = jnp.argsort(expert_flat, stable=True)
    e_sorted = expert_flat[order]
    group_start = jnp.concatenate(
        [jnp.zeros((1,), jnp.int32), jnp.cumsum(counts)[:-1]])
    padded = ((counts + _TM - 1) // _TM) * _TM
    padded_off = jnp.concatenate(
        [jnp.zeros((1,), jnp.int32), jnp.cumsum(padded)[:-1]])
    within = jnp.arange(t * _K, dtype=jnp.int32) - group_start[e_sorted]
    dest = padded_off[e_sorted] + within
    row_token = jnp.zeros((mp,), jnp.int32).at[dest].set(token_flat[order])
    row_gate = jnp.zeros((mp,), jnp.float32).at[dest].set(gate_flat[order])
    pos = jnp.zeros((t * _K,), jnp.int32).at[order].set(dest)
    pos = pos.reshape(t, _K)
    ends = jnp.cumsum(padded)
    tile_starts = jnp.arange(mp // _TM, dtype=jnp.int32) * _TM
    e_tile = jnp.clip(
        jnp.searchsorted(ends, tile_starts, side="right"), 0, e - 1
    ).astype(jnp.int32)

    xg = _make_gather(t, d, mp)(xf, row_token)
    y = _gmm(xg, W1, b1, W2, b2, row_gate.reshape(mp, 1), e_tile)
    out = _make_combine(t, do, mp)(y, pos[:, 0], pos[:, 1])

    counts_f = counts.astype(jnp.float32)
    total = jnp.sum(counts_f)
    fracs = counts_f / (total + 1e-08)
    lb_loss = _LBW * jnp.sum((fracs - 1.0 / e) ** 2)
    return out.reshape(b, s, do), lb_loss


# speedup vs baseline: 1.1717x; 1.1717x over previous
import functools

import jax
import jax.numpy as jnp
from jax import lax
from jax.experimental import pallas as pl
from jax.experimental.pallas import tpu as pltpu
from jax.experimental.pallas import tpu_sc as plsc

_K = 2
_LBW = 0.01
_TM = 512
_HB = 1024
_TMG = 1024
_NW = 32
_GCH = 16
_CCH = 8


def _gate_body(x_ref, wg_ref, bg_ref, i1_ref, i2_ref, g1_ref, g2_ref, cnt_ref):
    e = wg_ref.shape[0]
    logits = jnp.dot(x_ref[...], wg_ref[...].T,
                     preferred_element_type=jnp.float32) + bg_ref[0]
    iota = lax.broadcasted_iota(jnp.int32, logits.shape, 1)
    m1 = jnp.max(logits, axis=1, keepdims=True)
    i1 = jnp.min(jnp.where(logits == m1, iota, e), axis=1, keepdims=True)
    masked = jnp.where(iota == i1, -jnp.inf, logits)
    m2 = jnp.max(masked, axis=1, keepdims=True)
    i2 = jnp.min(jnp.where(masked == m2, iota, e), axis=1, keepdims=True)
    e2 = jnp.exp(m2 - m1)
    den = 1.0 + e2
    i1_ref[...] = i1
    i2_ref[...] = i2
    g1_ref[...] = 1.0 / den
    g2_ref[...] = e2 / den
    cnt = jnp.sum((iota == i1).astype(jnp.int32) + (iota == i2).astype(jnp.int32),
                  axis=0)
    cnt_ref[...] = cnt.reshape(1, 1, e)


def _gating(xf, Wg, bg):
    t, d = xf.shape
    e = Wg.shape[0]
    nt = t // _TMG
    out_shapes = (
        jax.ShapeDtypeStruct((t, 1), jnp.int32),
        jax.ShapeDtypeStruct((t, 1), jnp.int32),
        jax.ShapeDtypeStruct((t, 1), jnp.float32),
        jax.ShapeDtypeStruct((t, 1), jnp.float32),
        jax.ShapeDtypeStruct((nt, 1, e), jnp.int32),
    )
    col = pl.BlockSpec((_TMG, 1), lambda i: (i, 0))
    return pl.pallas_call(
        _gate_body,
        grid=(nt,),
        in_specs=[
            pl.BlockSpec((_TMG, d), lambda i: (i, 0)),
            pl.BlockSpec((e, d), lambda i: (0, 0)),
            pl.BlockSpec((1, e), lambda i: (0, 0)),
        ],
        out_specs=(col, col, col, col,
                   pl.BlockSpec((1, 1, e), lambda i: (i, 0, 0))),
        out_shape=out_shapes,
    )(xf, Wg, bg.reshape(1, e))


def _make_gather(t, d, mp):
    rows_per_w = mp // _NW
    n_pairs = rows_per_w // (2 * _GCH)
    mesh = plsc.VectorSubcoreMesh(core_axis_name="c", subcore_axis_name="s")

    @functools.partial(
        pl.kernel,
        mesh=mesh,
        out_type=jax.ShapeDtypeStruct((mp, d), jnp.float32),
        scratch_types=[
            pltpu.VMEM((rows_per_w,), jnp.int32),
            pltpu.VMEM((_GCH, d), jnp.float32),
            pltpu.VMEM((_GCH, d), jnp.float32),
            pltpu.SemaphoreType.DMA,
            pltpu.SemaphoreType.DMA,
        ],
    )
    def gather(tab_hbm, idx_hbm, out_hbm, idx_v, rows0, rows1, sem0, sem1):
        wid = lax.axis_index("s") * 2 + lax.axis_index("c")
        base = wid * rows_per_w
        pltpu.sync_copy(idx_hbm.at[pl.ds(base, rows_per_w)], idx_v)

        def start(c, rows, sem):
            pltpu.async_copy(
                tab_hbm.at[idx_v.at[pl.ds(c * _GCH, _GCH)]], rows, sem)

        def drain(rows, sem):
            pltpu.make_async_copy(tab_hbm.at[pl.ds(0, _GCH)], rows, sem).wait()

        def emit(c, rows):
            pltpu.sync_copy(rows, out_hbm.at[pl.ds(base + c * _GCH, _GCH)])

        start(0, rows0, sem0)

        def body(g, carry):
            c0 = 2 * g
            start(c0 + 1, rows1, sem1)
            drain(rows0, sem0)
            emit(c0, rows0)

            @pl.when(g + 1 < n_pairs)
            def _():
                start(c0 + 2, rows0, sem0)

            drain(rows1, sem1)
            emit(c0 + 1, rows1)
            return carry

        lax.fori_loop(0, n_pairs, body, 0)

    return gather


def _gmm_body(e_ref, xg_ref, w1_ref, w2_ref, b1_ref, b2_ref, g_ref, y_ref,
              acc_ref):
    j = pl.program_id(1)

    @pl.when(j == 0)
    def _():
        acc_ref[...] = jnp.zeros_like(acc_ref)

    xb = xg_ref[...]
    h = lax.dot_general(xb, w1_ref[0], (((1,), (1,)), ((), ())),
                        preferred_element_type=jnp.float32)
    h = jnp.maximum(h + b1_ref[0, 0], 0.0)
    acc_ref[...] += lax.dot_general(h, w2_ref[0], (((1,), (1,)), ((), ())),
                                    preferred_element_type=jnp.float32)

    @pl.when(j == pl.num_programs(1) - 1)
    def _():
        y_ref[...] = (acc_ref[...] + b2_ref[0, 0]) * g_ref[...]


def _gmm(xg, W1, b1, W2, b2, row_gate, e_tile):
    mp, d = xg.shape
    e, h, _ = W1.shape
    do = W2.shape[1]
    n_tiles = mp // _TM
    nh = h // _HB
    grid_spec = pltpu.PrefetchScalarGridSpec(
        num_scalar_prefetch=1,
        grid=(n_tiles, nh),
        in_specs=[
            pl.BlockSpec((_TM, d), lambda i, j, er: (i, 0)),
            pl.BlockSpec((1, _HB, d), lambda i, j, er: (er[i], j, 0)),
            pl.BlockSpec((1, do, _HB), lambda i, j, er: (er[i], 0, j)),
            pl.BlockSpec((1, 1, 1, _HB), lambda i, j, er: (er[i], j, 0, 0)),
            pl.BlockSpec((1, 1, do), lambda i, j, er: (er[i], 0, 0)),
            pl.BlockSpec((_TM, 1), lambda i, j, er: (i, 0)),
        ],
        out_specs=pl.BlockSpec((_TM, do), lambda i, j, er: (i, 0)),
        scratch_shapes=[pltpu.VMEM((_TM, do), jnp.float32)],
    )
    return pl.pallas_call(
        _gmm_body,
        grid_spec=grid_spec,
        out_shape=jax.ShapeDtypeStruct((mp, do), jnp.float32),
    )(e_tile, xg, W1, W2,
      b1.reshape(e, h // _HB, 1, _HB), b2.reshape(e, 1, do), row_gate)


def _make_combine(t, do, mp):
    tok_per_w = t // _NW
    n_pairs = tok_per_w // (2 * _CCH)
    nv = do // 16
    mesh = plsc.VectorSubcoreMesh(core_axis_name="c", subcore_axis_name="s")

    @functools.partial(
        pl.kernel,
        mesh=mesh,
        out_type=jax.ShapeDtypeStruct((t, do), jnp.float32),
        scratch_types=[
            pltpu.VMEM((tok_per_w,), jnp.int32),
            pltpu.VMEM((tok_per_w,), jnp.int32),
            pltpu.VMEM((_CCH, do), jnp.float32),
            pltpu.VMEM((_CCH, do), jnp.float32),
            pltpu.VMEM((_CCH, do), jnp.float32),
            pltpu.VMEM((_CCH, do), jnp.float32),
            pltpu.SemaphoreType.DMA,
            pltpu.SemaphoreType.DMA,
            pltpu.SemaphoreType.DMA,
            pltpu.SemaphoreType.DMA,
        ],
    )
    def combine(y_hbm, p0_hbm, p1_hbm, out_hbm, p0_v, p1_v, ra0, rb0, ra1,
                rb1, sa0, sb0, sa1, sb1):
        wid = lax.axis_index("s") * 2 + lax.axis_index("c")
        base = wid * tok_per_w
        pltpu.sync_copy(p0_hbm.at[pl.ds(base, tok_per_w)], p0_v)
        pltpu.sync_copy(p1_hbm.at[pl.ds(base, tok_per_w)], p1_v)

        def start(c, ra, rb, sa, sb):
            toff = c * _CCH
            pltpu.async_copy(y_hbm.at[p0_v.at[pl.ds(toff, _CCH)]], ra, sa)
            pltpu.async_copy(y_hbm.at[p1_v.at[pl.ds(toff, _CCH)]], rb, sb)

        def process(c, ra, rb, sa, sb):
            pltpu.make_async_copy(y_hbm.at[pl.ds(0, _CCH)], ra, sa).wait()
            pltpu.make_async_copy(y_hbm.at[pl.ds(0, _CCH)], rb, sb).wait()
            for r in range(_CCH):
                def add_col(v, carry2):
                    sl = pl.ds(v * 16, 16)
                    ra[r, sl] = ra[r, sl] + rb[r, sl]
                    return carry2
                lax.fori_loop(0, nv, add_col, 0)
            pltpu.sync_copy(ra, out_hbm.at[pl.ds(base + c * _CCH, _CCH)])

        start(0, ra0, rb0, sa0, sb0)

        def body(g, carry):
            c0 = 2 * g
            start(c0 + 1, ra1, rb1, sa1, sb1)
            process(c0, ra0, rb0, sa0, sb0)

            @pl.when(g + 1 < n_pairs)
            def _():
                start(c0 + 2, ra0, rb0, sa0, sb0)

            process(c0 + 1, ra1, rb1, sa1, sb1)
            return carry

        lax.fori_loop(0, n_pairs, body, 0)

    return combine


def kernel(x, Wg, bg, W1, b1, W2, b2):
    b, s, d = x.shape
    t = b * s
    e, h, _ = W1.shape
    do = W2.shape[1]
    mp = t * _K + e * _TM

    xf = x.reshape(t, d)
    i1, i2, g1, g2, cnt_tiles = _gating(xf, Wg, bg)
    counts = jnp.sum(cnt_tiles, axis=(0, 1))

    expert_flat = jnp.concatenate([i1, i2], axis=1).reshape(-1)
    gate_flat = jnp.concatenate([g1, g2], axis=1).reshape(-1)
    token_flat = jnp.repeat(jnp.arange(t, dtype=jnp.int32), _K)
    order = jnp.argsort(expert_flat, stable=True)
    e_sorted = expert_flat[order]
    group_start = jnp.concatenate(
        [jnp.zeros((1,), jnp.int32), jnp.cumsum(counts)[:-1]])
    padded = ((counts + _TM - 1) // _TM) * _TM
    padded_off = jnp.concatenate(
        [jnp.zeros((1,), jnp.int32), jnp.cumsum(padded)[:-1]])
    within = jnp.arange(t * _K, dtype=jnp.int32) - group_start[e_sorted]
    dest = padded_off[e_sorted] + within
    row_token = jnp.zeros((mp,), jnp.int32).at[dest].set(token_flat[order])
    row_gate = jnp.zeros((mp,), jnp.float32).at[dest].set(gate_flat[order])
    pos = jnp.zeros((t * _K,), jnp.int32).at[order].set(dest)
    pos = pos.reshape(t, _K)
    ends = jnp.cumsum(padded)
    tile_starts = jnp.arange(mp // _TM, dtype=jnp.int32) * _TM
    e_tile = jnp.clip(
        jnp.searchsorted(ends, tile_starts, side="right"), 0, e - 1
    ).astype(jnp.int32)

    xg = _make_gather(t, d, mp)(xf, row_token)
    y = _gmm(xg, W1, b1, W2, b2, row_gate.reshape(mp, 1), e_tile)
    out = _make_combine(t, do, mp)(y, pos[:, 0], pos[:, 1])

    counts_f = counts.astype(jnp.float32)
    total = jnp.sum(counts_f)
    fracs = counts_f / (total + 1e-08)
    lb_loss = _LBW * jnp.sum((fracs - 1.0 / e) ** 2)
    return out.reshape(b, s, do), lb_loss
